# Initial kernel scaffold; baseline (speedup 1.0000x reference)
import jax
import jax.numpy as jnp
from jax.experimental import pallas as pl


def kernel(query, q_sub, q_rel, hidden, edges, nodes, rela_embed, Ws_attn, Wr_attn, Wqr_attn_W, Wqr_attn_b, W_attn):
    raise NotImplementedError("write your pallas kernel here")


# speedup vs baseline: 1.6686x; 1.6686x over previous
import functools

import jax
import jax.numpy as jnp
from jax import lax
from jax.experimental import pallas as pl
from jax.experimental.pallas import tpu as pltpu
from jax.experimental.pallas import tpu_sc as plsc

B = 8
N = 10000
D = 128
E = 320000
R = 32

NC = 2
NS = 16
NW = NC * NS
EPW = E // NW
K = 80
NCHUNK = EPW // K


def _node_table_body(a_ref, ws_ref, out_ref):
    a = a_ref[...]
    out_ref[:, :D] = a
    out_ref[:, D:] = jnp.dot(a, ws_ref[...], preferred_element_type=jnp.float32)


def _combo_table_body(rela_ref, q_ref, wr_ref, wq_ref, b_ref, out_ref):
    rela = rela_ref[...]
    rr = jnp.dot(rela, wr_ref[...], preferred_element_type=jnp.float32)
    qw = jnp.dot(q_ref[...], wq_ref[...], preferred_element_type=jnp.float32)
    qw = qw + b_ref[...]
    out_ref[:, :D] = jnp.broadcast_to(rela[:, None, :], (R, B, D)).reshape(R * B, D)
    out_ref[:, D:] = (rr[:, None, :] + qw[None, :, :]).reshape(R * B, D)


def _combo_idx_body(rel_ref, bat_ref, out_ref):
    out_ref[...] = rel_ref[...] * 8 + bat_ref[...]


def _edge_body(t_hbm, s_hbm, sub_hbm, c_hbm, w_hbm, out_hbm,
               sub_v, c_v, hrows_v, srows_v, out_v, w_v,
               sem_t, sem_s):
    wid = lax.axis_index("s") * NC + lax.axis_index("c")
    base = wid * EPW
    pltpu.sync_copy(w_hbm, w_v)
    ones16 = jnp.ones((16,), jnp.float32)

    def chunk_body(j, carry):
        cbase = base + j * K
        pltpu.sync_copy(sub_hbm.at[pl.ds(cbase, K)], sub_v)
        pltpu.sync_copy(c_hbm.at[pl.ds(cbase, K)], c_v)
        cp_t = pltpu.async_copy(t_hbm.at[sub_v], hrows_v, sem_t)
        cp_s = pltpu.async_copy(s_hbm.at[c_v], srows_v, sem_s)
        cp_t.wait()
        cp_s.wait()

        def edge_body(e, c2):
            acc = jnp.zeros((16,), jnp.float32)
            for k in range(D // 16):
                hs = hrows_v[e, pl.ds(D + k * 16, 16)]
                cc = srows_v[e, pl.ds(D + k * 16, 16)]
                acc = acc + jnp.maximum(hs + cc, 0.0) * w_v[pl.ds(k * 16, 16)]
            a = jnp.sum(acc)
            alpha = 1.0 / (1.0 + jnp.exp(-a * ones16))
            for k in range(D // 16):
                sl = pl.ds(k * 16, 16)
                out_v[e, sl] = hrows_v[e, sl] * srows_v[e, sl] * alpha
            return c2

        lax.fori_loop(0, K, edge_body, jnp.int32(0))
        pltpu.sync_copy(out_v, out_hbm.at[pl.ds(cbase, K)])
        return carry

    lax.fori_loop(0, NCHUNK, chunk_body, jnp.int32(0))


def kernel(query, q_sub, q_rel, hidden, edges, nodes, rela_embed,
           Ws_attn, Wr_attn, Wqr_attn_W, Wqr_attn_b, W_attn):
    all_ent = hidden.reshape(-1, D)
    blk = 640
    node_table = pl.pallas_call(
        _node_table_body,
        grid=(all_ent.shape[0] // blk,),
        in_specs=[
            pl.BlockSpec((blk, D), lambda i: (i, 0)),
            pl.BlockSpec((D, D), lambda i: (0, 0)),
        ],
        out_specs=pl.BlockSpec((blk, 2 * D), lambda i: (i, 0)),
        out_shape=jax.ShapeDtypeStruct((all_ent.shape[0], 2 * D), jnp.float32),
    )(all_ent, Ws_attn)

    combo_table = pl.pallas_call(
        _combo_table_body,
        out_shape=jax.ShapeDtypeStruct((R * B, 2 * D), jnp.float32),
    )(rela_embed, query, Wr_attn, Wqr_attn_W, Wqr_attn_b.reshape(1, D))

    rel2d = edges[:, 2].reshape(E // D, D)
    bat2d = edges[:, 0].reshape(E // D, D)
    combo_idx = pl.pallas_call(
        _combo_idx_body,
        out_shape=jax.ShapeDtypeStruct((E // D, D), jnp.int32),
    )(rel2d, bat2d).reshape(E)

    mesh = plsc.VectorSubcoreMesh(
        core_axis_name="c", subcore_axis_name="s",
        num_cores=NC, num_subcores=NS)
    sc = functools.partial(
        pl.kernel,
        mesh=mesh,
        compiler_params=pltpu.CompilerParams(needs_layout_passes=False),
        out_type=jax.ShapeDtypeStruct((E, D), jnp.float32),
        scratch_types=[
            pltpu.VMEM((K,), jnp.int32),
            pltpu.VMEM((K,), jnp.int32),
            pltpu.VMEM((K, 2 * D), jnp.float32),
            pltpu.VMEM((K, 2 * D), jnp.float32),
            pltpu.VMEM((K, D), jnp.float32),
            pltpu.VMEM((D,), jnp.float32),
            pltpu.SemaphoreType.DMA,
            pltpu.SemaphoreType.DMA,
        ],
    )(_edge_body)
    return sc(node_table, combo_table, edges[:, 1], combo_idx, W_attn.reshape(D))


# speedup vs baseline: 3.2477x; 1.9464x over previous
import functools

import jax
import jax.numpy as jnp
from jax import lax
from jax.experimental import pallas as pl
from jax.experimental.pallas import tpu as pltpu
from jax.experimental.pallas import tpu_sc as plsc

B = 8
N = 10000
D = 128
E = 320000
R = 32

NC = 2
NS = 16
NW = NC * NS
EPW = E // NW
K = 80
NCHUNK = EPW // K


def _node_table_body(a_ref, ws_ref, out_ref):
    a = a_ref[...]
    out_ref[:, :D] = a
    out_ref[:, D:] = jnp.dot(a, ws_ref[...], preferred_element_type=jnp.float32)


def _combo_table_body(rela_ref, q_ref, wr_ref, wq_ref, b_ref, out_ref):
    rela = rela_ref[...]
    rr = jnp.dot(rela, wr_ref[...], preferred_element_type=jnp.float32)
    qw = jnp.dot(q_ref[...], wq_ref[...], preferred_element_type=jnp.float32)
    qw = qw + b_ref[...]
    out_ref[:, :D] = jnp.broadcast_to(rela[:, None, :], (R, B, D)).reshape(R * B, D)
    out_ref[:, D:] = (rr[:, None, :] + qw[None, :, :]).reshape(R * B, D)


def _combo_idx_body(rel_ref, bat_ref, out_ref):
    out_ref[...] = rel_ref[...] * 8 + bat_ref[...]


def _edge_body(t_hbm, s_hbm, sub_hbm, c_hbm, w_hbm, out_hbm,
               sub_v, c_v, hrows_v, srows_v, out_v, w_v,
               sem_t, sem_s):
    wid = lax.axis_index("s") * NC + lax.axis_index("c")
    base = wid * EPW
    pltpu.sync_copy(w_hbm, w_v)
    ones16 = jnp.ones((16,), jnp.float32)
    wk = [w_v[pl.ds(k * 16, 16)] for k in range(D // 16)]

    def chunk_body(j, carry):
        cbase = base + j * K
        pltpu.sync_copy(sub_hbm.at[pl.ds(cbase, K)], sub_v)
        pltpu.sync_copy(c_hbm.at[pl.ds(cbase, K)], c_v)
        cp_t = pltpu.async_copy(t_hbm.at[sub_v], hrows_v, sem_t)
        cp_s = pltpu.async_copy(s_hbm.at[c_v], srows_v, sem_s)
        cp_t.wait()
        cp_s.wait()

        @plsc.parallel_loop(0, K, 1, unroll=4)
        def edge_body(e):
            acc = jnp.zeros((16,), jnp.float32)
            for k in range(D // 16):
                hs = hrows_v[e, pl.ds(D + k * 16, 16)]
                cc = srows_v[e, pl.ds(D + k * 16, 16)]
                acc = acc + jnp.maximum(hs + cc, 0.0) * wk[k]
            a = jnp.sum(acc)
            alpha = 1.0 / (1.0 + jnp.exp(-a * ones16))
            for k in range(D // 16):
                sl = pl.ds(k * 16, 16)
                out_v[e, sl] = hrows_v[e, sl] * srows_v[e, sl] * alpha

        pltpu.sync_copy(out_v, out_hbm.at[pl.ds(cbase, K)])
        return carry

    lax.fori_loop(0, NCHUNK, chunk_body, jnp.int32(0))


def kernel(query, q_sub, q_rel, hidden, edges, nodes, rela_embed,
           Ws_attn, Wr_attn, Wqr_attn_W, Wqr_attn_b, W_attn):
    all_ent = hidden.reshape(-1, D)
    blk = 640
    node_table = pl.pallas_call(
        _node_table_body,
        grid=(all_ent.shape[0] // blk,),
        in_specs=[
            pl.BlockSpec((blk, D), lambda i: (i, 0)),
            pl.BlockSpec((D, D), lambda i: (0, 0)),
        ],
        out_specs=pl.BlockSpec((blk, 2 * D), lambda i: (i, 0)),
        out_shape=jax.ShapeDtypeStruct((all_ent.shape[0], 2 * D), jnp.float32),
    )(all_ent, Ws_attn)

    combo_table = pl.pallas_call(
        _combo_table_body,
        out_shape=jax.ShapeDtypeStruct((R * B, 2 * D), jnp.float32),
    )(rela_embed, query, Wr_attn, Wqr_attn_W, Wqr_attn_b.reshape(1, D))

    rel2d = edges[:, 2].reshape(E // D, D)
    bat2d = edges[:, 0].reshape(E // D, D)
    combo_idx = pl.pallas_call(
        _combo_idx_body,
        out_shape=jax.ShapeDtypeStruct((E // D, D), jnp.int32),
    )(rel2d, bat2d).reshape(E)

    mesh = plsc.VectorSubcoreMesh(
        core_axis_name="c", subcore_axis_name="s",
        num_cores=NC, num_subcores=NS)
    sc = functools.partial(
        pl.kernel,
        mesh=mesh,
        compiler_params=pltpu.CompilerParams(needs_layout_passes=False),
        out_type=jax.ShapeDtypeStruct((E, D), jnp.float32),
        scratch_types=[
            pltpu.VMEM((K,), jnp.int32),
            pltpu.VMEM((K,), jnp.int32),
            pltpu.VMEM((K, 2 * D), jnp.float32),
            pltpu.VMEM((K, 2 * D), jnp.float32),
            pltpu.VMEM((K, D), jnp.float32),
            pltpu.VMEM((D,), jnp.float32),
            pltpu.SemaphoreType.DMA,
            pltpu.SemaphoreType.DMA,
        ],
    )(_edge_body)
    return sc(node_table, combo_table, edges[:, 1], combo_idx, W_attn.reshape(D))


# speedup vs baseline: 4.0400x; 1.2440x over previous
import functools

import jax
import jax.numpy as jnp
from jax import lax
from jax.experimental import pallas as pl
from jax.experimental.pallas import tpu as pltpu
from jax.experimental.pallas import tpu_sc as plsc

B = 8
N = 10000
D = 128
E = 320000
R = 32

NC = 2
NS = 16
NW = NC * NS
EPW = E // NW
K = 40
NCHUNK = EPW // K
HALF = NCHUNK // 2


def _node_table_body(a_ref, ws_ref, out_ref):
    a = a_ref[...]
    out_ref[:, :D] = a
    out_ref[:, D:] = jnp.dot(a, ws_ref[...], preferred_element_type=jnp.float32)


def _combo_table_body(rela_ref, q_ref, wr_ref, wq_ref, b_ref, out_ref):
    rela = rela_ref[...]
    rr = jnp.dot(rela, wr_ref[...], preferred_element_type=jnp.float32)
    qw = jnp.dot(q_ref[...], wq_ref[...], preferred_element_type=jnp.float32)
    qw = qw + b_ref[...]
    out_ref[:, :D] = jnp.broadcast_to(rela[:, None, :], (R, B, D)).reshape(R * B, D)
    out_ref[:, D:] = (rr[:, None, :] + qw[None, :, :]).reshape(R * B, D)


def _combo_idx_body(rel_ref, bat_ref, out_ref):
    out_ref[...] = rel_ref[...] * 8 + bat_ref[...]


def _edge_body(t_hbm, s_hbm, sub_hbm, c_hbm, w_hbm, out_hbm,
               sub_all, c_all, hrows0, hrows1, srows0, srows1, out_v, w_v,
               sem_t0, sem_t1, sem_s0, sem_s1):
    wid = lax.axis_index("s") * NC + lax.axis_index("c")
    base = wid * EPW
    pltpu.sync_copy(w_hbm, w_v)
    pltpu.sync_copy(sub_hbm.at[pl.ds(base, EPW)], sub_all)
    pltpu.sync_copy(c_hbm.at[pl.ds(base, EPW)], c_all)
    ones16 = jnp.ones((16,), jnp.float32)
    wk = [w_v[pl.ds(k * 16, 16)] for k in range(D // 16)]
    hrows = (hrows0, hrows1)
    srows = (srows0, srows1)
    sem_t = (sem_t0, sem_t1)
    sem_s = (sem_s0, sem_s1)

    def issue(j, b):
        pltpu.async_copy(t_hbm.at[sub_all.at[pl.ds(j * K, K)]], hrows[b], sem_t[b])
        pltpu.async_copy(s_hbm.at[c_all.at[pl.ds(j * K, K)]], srows[b], sem_s[b])

    def wait(j, b):
        pltpu.make_async_copy(
            t_hbm.at[sub_all.at[pl.ds(j * K, K)]], hrows[b], sem_t[b]).wait()
        pltpu.make_async_copy(
            s_hbm.at[c_all.at[pl.ds(j * K, K)]], srows[b], sem_s[b]).wait()

    def compute(j, b):
        hv, sv = hrows[b], srows[b]

        @plsc.parallel_loop(0, K, 1, unroll=4)
        def edge_body(e):
            acc = jnp.zeros((16,), jnp.float32)
            for k in range(D // 16):
                hs = hv[e, pl.ds(D + k * 16, 16)]
                cc = sv[e, pl.ds(D + k * 16, 16)]
                acc = acc + jnp.maximum(hs + cc, 0.0) * wk[k]
            a = jnp.sum(acc)
            alpha = 1.0 / (1.0 + jnp.exp(-a * ones16))
            for k in range(D // 16):
                sl = pl.ds(k * 16, 16)
                out_v[e, sl] = hv[e, sl] * sv[e, sl] * alpha

        pltpu.sync_copy(out_v, out_hbm.at[pl.ds(base + j * K, K)])

    issue(jnp.int32(0), 0)

    def chunk_body(i, carry):
        j0 = 2 * i
        j1 = j0 + 1
        issue(j1, 1)
        wait(j0, 0)
        compute(j0, 0)

        @pl.when(i < HALF - 1)
        def _():
            issue(j0 + 2, 0)

        wait(j1, 1)
        compute(j1, 1)
        return carry

    lax.fori_loop(0, HALF, chunk_body, jnp.int32(0))


def kernel(query, q_sub, q_rel, hidden, edges, nodes, rela_embed,
           Ws_attn, Wr_attn, Wqr_attn_W, Wqr_attn_b, W_attn):
    all_ent = hidden.reshape(-1, D)
    blk = 640
    node_table = pl.pallas_call(
        _node_table_body,
        grid=(all_ent.shape[0] // blk,),
        in_specs=[
            pl.BlockSpec((blk, D), lambda i: (i, 0)),
            pl.BlockSpec((D, D), lambda i: (0, 0)),
        ],
        out_specs=pl.BlockSpec((blk, 2 * D), lambda i: (i, 0)),
        out_shape=jax.ShapeDtypeStruct((all_ent.shape[0], 2 * D), jnp.float32),
    )(all_ent, Ws_attn)

    combo_table = pl.pallas_call(
        _combo_table_body,
        out_shape=jax.ShapeDtypeStruct((R * B, 2 * D), jnp.float32),
    )(rela_embed, query, Wr_attn, Wqr_attn_W, Wqr_attn_b.reshape(1, D))

    rel2d = edges[:, 2].reshape(E // D, D)
    bat2d = edges[:, 0].reshape(E // D, D)
    combo_idx = pl.pallas_call(
        _combo_idx_body,
        out_shape=jax.ShapeDtypeStruct((E // D, D), jnp.int32),
    )(rel2d, bat2d).reshape(E)

    mesh = plsc.VectorSubcoreMesh(
        core_axis_name="c", subcore_axis_name="s",
        num_cores=NC, num_subcores=NS)
    sc = functools.partial(
        pl.kernel,
        mesh=mesh,
        compiler_params=pltpu.CompilerParams(needs_layout_passes=False),
        out_type=jax.ShapeDtypeStruct((E, D), jnp.float32),
        scratch_types=[
            pltpu.VMEM((EPW,), jnp.int32),
            pltpu.VMEM((EPW,), jnp.int32),
            pltpu.VMEM((K, 2 * D), jnp.float32),
            pltpu.VMEM((K, 2 * D), jnp.float32),
            pltpu.VMEM((K, 2 * D), jnp.float32),
            pltpu.VMEM((K, 2 * D), jnp.float32),
            pltpu.VMEM((K, D), jnp.float32),
            pltpu.VMEM((D,), jnp.float32),
            pltpu.SemaphoreType.DMA,
            pltpu.SemaphoreType.DMA,
            pltpu.SemaphoreType.DMA,
            pltpu.SemaphoreType.DMA,
        ],
    )(_edge_body)
    return sc(node_table, combo_table, edges[:, 1], combo_idx, W_attn.reshape(D))


# speedup vs baseline: 4.0502x; 1.0025x over previous
import functools

import jax
import jax.numpy as jnp
from jax import lax
from jax.experimental import pallas as pl
from jax.experimental.pallas import tpu as pltpu
from jax.experimental.pallas import tpu_sc as plsc

B = 8
N = 10000
D = 128
E = 320000
R = 32

NC = 2
NS = 16
NW = NC * NS
EPW = E // NW
K = 40
NCHUNK = EPW // K
HALF = NCHUNK // 2


def _node_table_body(a_ref, ws_ref, out_ref):
    a = a_ref[...]
    out_ref[:, :D] = a
    out_ref[:, D:] = jnp.dot(a, ws_ref[...], preferred_element_type=jnp.float32)


def _combo_table_body(rela_ref, q_ref, wr_ref, wq_ref, b_ref, out_ref):
    rela = rela_ref[...]
    rr = jnp.dot(rela, wr_ref[...], preferred_element_type=jnp.float32)
    qw = jnp.dot(q_ref[...], wq_ref[...], preferred_element_type=jnp.float32)
    qw = qw + b_ref[...]
    out_ref[:, :D] = jnp.broadcast_to(rela[:, None, :], (R, B, D)).reshape(R * B, D)
    out_ref[:, D:] = (rr[:, None, :] + qw[None, :, :]).reshape(R * B, D)


def _combo_idx_body(rel_ref, bat_ref, out_ref):
    out_ref[...] = rel_ref[...] * 8 + bat_ref[...]


def _edge_body(t_hbm, s_hbm, sub_hbm, c_hbm, w_hbm, out_hbm,
               sub_all, c_all, hrows0, hrows1, srows0, srows1, out0, out1, w_v,
               sem_t0, sem_t1, sem_s0, sem_s1, sem_o0, sem_o1):
    wid = lax.axis_index("s") * NC + lax.axis_index("c")
    base = wid * EPW
    pltpu.sync_copy(w_hbm, w_v)
    pltpu.sync_copy(sub_hbm.at[pl.ds(base, EPW)], sub_all)
    pltpu.sync_copy(c_hbm.at[pl.ds(base, EPW)], c_all)
    ones16 = jnp.ones((16,), jnp.float32)
    wk = [w_v[pl.ds(k * 16, 16)] for k in range(D // 16)]
    hrows = (hrows0, hrows1)
    srows = (srows0, srows1)
    outs = (out0, out1)
    sem_t = (sem_t0, sem_t1)
    sem_s = (sem_s0, sem_s1)
    sem_o = (sem_o0, sem_o1)

    def issue(j, b):
        pltpu.async_copy(t_hbm.at[sub_all.at[pl.ds(j * K, K)]], hrows[b], sem_t[b])
        pltpu.async_copy(s_hbm.at[c_all.at[pl.ds(j * K, K)]], srows[b], sem_s[b])

    def wait(j, b):
        pltpu.make_async_copy(
            t_hbm.at[sub_all.at[pl.ds(j * K, K)]], hrows[b], sem_t[b]).wait()
        pltpu.make_async_copy(
            s_hbm.at[c_all.at[pl.ds(j * K, K)]], srows[b], sem_s[b]).wait()

    def compute(j, b):
        hv, sv, ov = hrows[b], srows[b], outs[b]

        @pl.when(j >= 2)
        def _():
            pltpu.make_async_copy(
                ov, out_hbm.at[pl.ds(base + (j - 2) * K, K)], sem_o[b]).wait()

        @plsc.parallel_loop(0, K, 1, unroll=8)
        def edge_body(e):
            acc = jnp.zeros((16,), jnp.float32)
            for k in range(D // 16):
                hs = hv[e, pl.ds(D + k * 16, 16)]
                cc = sv[e, pl.ds(D + k * 16, 16)]
                acc = acc + jnp.maximum(hs + cc, 0.0) * wk[k]
            a = jnp.sum(acc)
            alpha = 1.0 / (1.0 + jnp.exp(-a * ones16))
            for k in range(D // 16):
                sl = pl.ds(k * 16, 16)
                ov[e, sl] = hv[e, sl] * sv[e, sl] * alpha

        pltpu.async_copy(ov, out_hbm.at[pl.ds(base + j * K, K)], sem_o[b])

    issue(jnp.int32(0), 0)

    def chunk_body(i, carry):
        j0 = 2 * i
        j1 = j0 + 1
        issue(j1, 1)
        wait(j0, 0)
        compute(j0, 0)

        @pl.when(i < HALF - 1)
        def _():
            issue(j0 + 2, 0)

        wait(j1, 1)
        compute(j1, 1)
        return carry

    lax.fori_loop(0, HALF, chunk_body, jnp.int32(0))
    pltpu.make_async_copy(
        out0, out_hbm.at[pl.ds(base + (NCHUNK - 2) * K, K)], sem_o0).wait()
    pltpu.make_async_copy(
        out1, out_hbm.at[pl.ds(base + (NCHUNK - 1) * K, K)], sem_o1).wait()


def kernel(query, q_sub, q_rel, hidden, edges, nodes, rela_embed,
           Ws_attn, Wr_attn, Wqr_attn_W, Wqr_attn_b, W_attn):
    all_ent = hidden.reshape(-1, D)
    blk = 640
    node_table = pl.pallas_call(
        _node_table_body,
        grid=(all_ent.shape[0] // blk,),
        in_specs=[
            pl.BlockSpec((blk, D), lambda i: (i, 0)),
            pl.BlockSpec((D, D), lambda i: (0, 0)),
        ],
        out_specs=pl.BlockSpec((blk, 2 * D), lambda i: (i, 0)),
        out_shape=jax.ShapeDtypeStruct((all_ent.shape[0], 2 * D), jnp.float32),
    )(all_ent, Ws_attn)

    combo_table = pl.pallas_call(
        _combo_table_body,
        out_shape=jax.ShapeDtypeStruct((R * B, 2 * D), jnp.float32),
    )(rela_embed, query, Wr_attn, Wqr_attn_W, Wqr_attn_b.reshape(1, D))

    rel2d = edges[:, 2].reshape(E // D, D)
    bat2d = edges[:, 0].reshape(E // D, D)
    combo_idx = pl.pallas_call(
        _combo_idx_body,
        out_shape=jax.ShapeDtypeStruct((E // D, D), jnp.int32),
    )(rel2d, bat2d).reshape(E)

    mesh = plsc.VectorSubcoreMesh(
        core_axis_name="c", subcore_axis_name="s",
        num_cores=NC, num_subcores=NS)
    sc = functools.partial(
        pl.kernel,
        mesh=mesh,
        compiler_params=pltpu.CompilerParams(needs_layout_passes=False),
        out_type=jax.ShapeDtypeStruct((E, D), jnp.float32),
        scratch_types=[
            pltpu.VMEM((EPW,), jnp.int32),
            pltpu.VMEM((EPW,), jnp.int32),
            pltpu.VMEM((K, 2 * D), jnp.float32),
            pltpu.VMEM((K, 2 * D), jnp.float32),
            pltpu.VMEM((K, 2 * D), jnp.float32),
            pltpu.VMEM((K, 2 * D), jnp.float32),
            pltpu.VMEM((K, D), jnp.float32),
            pltpu.VMEM((K, D), jnp.float32),
            pltpu.VMEM((D,), jnp.float32),
            pltpu.SemaphoreType.DMA,
            pltpu.SemaphoreType.DMA,
            pltpu.SemaphoreType.DMA,
            pltpu.SemaphoreType.DMA,
            pltpu.SemaphoreType.DMA,
            pltpu.SemaphoreType.DMA,
        ],
    )(_edge_body)
    return sc(node_table, combo_table, edges[:, 1], combo_idx, W_attn.reshape(D))


# speedup vs baseline: 4.4915x; 1.1090x over previous
import functools

import jax
import jax.numpy as jnp
from jax import lax
from jax.experimental import pallas as pl
from jax.experimental.pallas import tpu as pltpu
from jax.experimental.pallas import tpu_sc as plsc

B = 8
N = 10000
D = 128
E = 320000
R = 32

NC = 2
NS = 16
NW = NC * NS
EPW = E // NW
K = 40
NCHUNK = EPW // K
HALF = NCHUNK // 2


def _node_table_body(a_ref, ws_ref, out_ref):
    a = a_ref[...]
    out_ref[:, :D] = a
    out_ref[:, D:] = jnp.dot(a, ws_ref[...], preferred_element_type=jnp.float32)


def _combo_table_body(rela_ref, q_ref, wr_ref, wq_ref, b_ref, out_ref):
    rela = rela_ref[...]
    rr = jnp.dot(rela, wr_ref[...], preferred_element_type=jnp.float32)
    qw = jnp.dot(q_ref[...], wq_ref[...], preferred_element_type=jnp.float32)
    qw = qw + b_ref[...]
    out_ref[:, :D] = jnp.broadcast_to(rela[:, None, :], (R, B, D)).reshape(R * B, D)
    out_ref[:, D:] = (rr[:, None, :] + qw[None, :, :]).reshape(R * B, D)


def _combo_idx_body(rel_ref, bat_ref, out_ref):
    out_ref[...] = rel_ref[...] * 8 + bat_ref[...]


def _edge_body(t_hbm, s_hbm, sub_hbm, c_hbm, w_hbm, out_hbm,
               sub_all, c_all, s_v, hrows0, hrows1, out0, out1, w_v,
               sem_t0, sem_t1, sem_o0, sem_o1):
    wid = lax.axis_index("s") * NC + lax.axis_index("c")
    base = wid * EPW
    pltpu.sync_copy(w_hbm, w_v)
    pltpu.sync_copy(s_hbm, s_v)
    pltpu.sync_copy(sub_hbm.at[pl.ds(base, EPW)], sub_all)
    pltpu.sync_copy(c_hbm.at[pl.ds(base, EPW)], c_all.at[pl.ds(0, EPW)])
    ones16 = jnp.ones((16,), jnp.float32)
    wk = [w_v[pl.ds(k * 16, 16)] for k in range(D // 16)]
    hrows = (hrows0, hrows1)
    outs = (out0, out1)
    sem_t = (sem_t0, sem_t1)
    sem_o = (sem_o0, sem_o1)

    def issue(j, b):
        pltpu.async_copy(t_hbm.at[sub_all.at[pl.ds(j * K, K)]], hrows[b], sem_t[b])

    def wait(j, b):
        pltpu.make_async_copy(
            t_hbm.at[sub_all.at[pl.ds(j * K, K)]], hrows[b], sem_t[b]).wait()

    def compute(j, b):
        hv, ov = hrows[b], outs[b]
        ebase = j * K

        @pl.when(j >= 2)
        def _():
            pltpu.make_async_copy(
                ov, out_hbm.at[pl.ds(base + (j - 2) * K, K)], sem_o[b]).wait()

        @plsc.parallel_loop(0, K, 1, unroll=8)
        def edge_body(e):
            ce = c_all[pl.ds(ebase + e, 16)][0]
            acc = jnp.zeros((16,), jnp.float32)
            for k in range(D // 16):
                hs = hv[e, pl.ds(D + k * 16, 16)]
                cc = s_v[ce, pl.ds(D + k * 16, 16)]
                acc = acc + jnp.maximum(hs + cc, 0.0) * wk[k]
            a = jnp.sum(acc)
            alpha = 1.0 / (1.0 + jnp.exp(-a * ones16))
            for k in range(D // 16):
                sl = pl.ds(k * 16, 16)
                ov[e, sl] = hv[e, sl] * s_v[ce, sl] * alpha

        pltpu.async_copy(ov, out_hbm.at[pl.ds(base + j * K, K)], sem_o[b])

    issue(jnp.int32(0), 0)

    def chunk_body(i, carry):
        j0 = 2 * i
        j1 = j0 + 1
        issue(j1, 1)
        wait(j0, 0)
        compute(j0, 0)

        @pl.when(i < HALF - 1)
        def _():
            issue(j0 + 2, 0)

        wait(j1, 1)
        compute(j1, 1)
        return carry

    lax.fori_loop(0, HALF, chunk_body, jnp.int32(0))
    pltpu.make_async_copy(
        out0, out_hbm.at[pl.ds(base + (NCHUNK - 2) * K, K)], sem_o0).wait()
    pltpu.make_async_copy(
        out1, out_hbm.at[pl.ds(base + (NCHUNK - 1) * K, K)], sem_o1).wait()


def kernel(query, q_sub, q_rel, hidden, edges, nodes, rela_embed,
           Ws_attn, Wr_attn, Wqr_attn_W, Wqr_attn_b, W_attn):
    all_ent = hidden.reshape(-1, D)
    blk = 640
    node_table = pl.pallas_call(
        _node_table_body,
        grid=(all_ent.shape[0] // blk,),
        in_specs=[
            pl.BlockSpec((blk, D), lambda i: (i, 0)),
            pl.BlockSpec((D, D), lambda i: (0, 0)),
        ],
        out_specs=pl.BlockSpec((blk, 2 * D), lambda i: (i, 0)),
        out_shape=jax.ShapeDtypeStruct((all_ent.shape[0], 2 * D), jnp.float32),
    )(all_ent, Ws_attn)

    combo_table = pl.pallas_call(
        _combo_table_body,
        out_shape=jax.ShapeDtypeStruct((R * B, 2 * D), jnp.float32),
    )(rela_embed, query, Wr_attn, Wqr_attn_W, Wqr_attn_b.reshape(1, D))

    rel2d = edges[:, 2].reshape(E // D, D)
    bat2d = edges[:, 0].reshape(E // D, D)
    combo_idx = pl.pallas_call(
        _combo_idx_body,
        out_shape=jax.ShapeDtypeStruct((E // D, D), jnp.int32),
    )(rel2d, bat2d).reshape(E)

    mesh = plsc.VectorSubcoreMesh(
        core_axis_name="c", subcore_axis_name="s",
        num_cores=NC, num_subcores=NS)
    sc = functools.partial(
        pl.kernel,
        mesh=mesh,
        compiler_params=pltpu.CompilerParams(needs_layout_passes=False),
        out_type=jax.ShapeDtypeStruct((E, D), jnp.float32),
        scratch_types=[
            pltpu.VMEM((EPW,), jnp.int32),
            pltpu.VMEM((EPW + 16,), jnp.int32),
            pltpu.VMEM((R * B, 2 * D), jnp.float32),
            pltpu.VMEM((K, 2 * D), jnp.float32),
            pltpu.VMEM((K, 2 * D), jnp.float32),
            pltpu.VMEM((K, D), jnp.float32),
            pltpu.VMEM((K, D), jnp.float32),
            pltpu.VMEM((D,), jnp.float32),
            pltpu.SemaphoreType.DMA,
            pltpu.SemaphoreType.DMA,
            pltpu.SemaphoreType.DMA,
            pltpu.SemaphoreType.DMA,
        ],
    )(_edge_body)
    return sc(node_table, combo_table, edges[:, 1], combo_idx, W_attn.reshape(D))
